# baseline (speedup 1.0000x reference)
import functools

import jax
import jax.numpy as jnp
from jax import lax
from jax.experimental import pallas as pl
from jax.experimental.pallas import tpu as pltpu
from jax.experimental.pallas import tpu_sc as plsc


def _kernel_sc(x, pos_table):
    B, S, D = x.shape
    info = plsc.get_sparse_core_info()
    NC, NS, L = info.num_cores, info.num_subcores, info.num_lanes
    NW = NC * NS
    RW = S // NW
    C = 32
    NCH = RW // C
    NI = B * NCH
    NB = 2

    mesh = plsc.VectorSubcoreMesh(core_axis_name="c", subcore_axis_name="s")

    @functools.partial(
        pl.kernel,
        mesh=mesh,
        out_type=jax.ShapeDtypeStruct((B, S, D), jnp.float32),
        compiler_params=pltpu.CompilerParams(use_tc_tiling_on_sc=True),
        scratch_types=[pltpu.VMEM((C, D), jnp.float32)]
        + [pltpu.VMEM((C, D), jnp.float32) for _ in range(NB)]
        + [pltpu.SemaphoreType.DMA for _ in range(2 * NB + 1)],
    )
    def run(x_hbm, pos_hbm, out_hbm, pos_v, *rest):
        bufs = rest[:NB]
        lsems = rest[NB : 2 * NB]
        ssems = rest[2 * NB : 3 * NB]
        psem = rest[3 * NB]

        wid = lax.axis_index("s") * NC + lax.axis_index("c")
        r0 = wid * RW

        def item_cb(i):
            return i // B, i % B

        def load_pos(c):
            return pltpu.async_copy(
                pos_hbm.at[pl.ds(r0 + c * C, C), :], pos_v, psem
            )

        def load(i, k):
            c, b = item_cb(i)
            return pltpu.async_copy(
                x_hbm.at[b, pl.ds(r0 + c * C, C), :], bufs[k], lsems[k]
            )

        pdesc = load_pos(0)
        ldesc = [None] * NB
        sdesc = [None] * NB
        ldesc[0] = load(0, 0)

        for i in range(NI):
            k = i % NB
            c, b = item_cb(i)
            ldesc[k].wait()
            if b == 0:
                pdesc.wait()

            buf = bufs[k]

            @plsc.parallel_loop(0, C * D, L, unroll=8)
            def add_body(j):
                r = j // D
                col = j % D
                plsc.addupdate(
                    buf.at[r, pl.ds(col, L)], pos_v[r, pl.ds(col, L)]
                )

            if b == B - 1 and c + 1 < NCH:
                pdesc = load_pos(c + 1)

            sdesc[k] = pltpu.async_copy(
                buf, out_hbm.at[b, pl.ds(r0 + c * C, C), :], ssems[k]
            )

            ni = i + 1
            if ni < NI:
                nk = ni % NB
                if sdesc[nk] is not None:
                    sdesc[nk].wait()
                ldesc[nk] = load(ni, nk)

        for i in range(max(0, NI - NB), NI):
            sdesc[i % NB].wait()

    return run(x, pos_table)


def kernel(x, pos_table):
    return _kernel_sc(x, pos_table)


# speedup vs baseline: 1.2296x; 1.2296x over previous
import functools

import jax
import jax.numpy as jnp
from jax import lax
from jax.experimental import pallas as pl
from jax.experimental.pallas import tpu as pltpu
from jax.experimental.pallas import tpu_sc as plsc


def _kernel_sc(x, pos_table):
    B, S, D = x.shape
    info = plsc.get_sparse_core_info()
    NC, NS, L = info.num_cores, info.num_subcores, info.num_lanes
    NW = NC * NS
    RW = S // NW
    C = 16
    NCH = RW // C
    PCH = 2
    NPH = NCH // PCH
    NI = B * NCH
    NB = 4
    LOOKAHEAD = 2

    mesh = plsc.VectorSubcoreMesh(core_axis_name="c", subcore_axis_name="s")

    @functools.partial(
        pl.kernel,
        mesh=mesh,
        out_type=jax.ShapeDtypeStruct((B, S, D), jnp.float32),
        compiler_params=pltpu.CompilerParams(use_tc_tiling_on_sc=True),
        scratch_types=[pltpu.VMEM((PCH * C, D), jnp.float32)]
        + [pltpu.VMEM((C, D), jnp.float32) for _ in range(NB)]
        + [pltpu.SemaphoreType.DMA for _ in range(2 * NB + 1)],
    )
    def run(x_hbm, pos_hbm, out_hbm, pos_v, *rest):
        bufs = rest[:NB]
        lsems = rest[NB : 2 * NB]
        ssems = rest[2 * NB : 3 * NB]
        psem = rest[3 * NB]

        wid = lax.axis_index("s") * NC + lax.axis_index("c")
        r0 = wid * RW

        def item_cb(i):
            p, r = i // (B * PCH), i % (B * PCH)
            b, cc = r // PCH, r % PCH
            return p * PCH + cc, b

        def load_pos(p):
            return pltpu.async_copy(
                pos_hbm.at[pl.ds(r0 + p * PCH * C, PCH * C), :], pos_v, psem
            )

        def load(i, k):
            c, b = item_cb(i)
            return pltpu.async_copy(
                x_hbm.at[b, pl.ds(r0 + c * C, C), :], bufs[k], lsems[k]
            )

        pdesc = load_pos(0)
        ldesc = [None] * NB
        sdesc = [None] * NB
        for k in range(LOOKAHEAD):
            ldesc[k] = load(k, k)

        for i in range(NI):
            k = i % NB
            c, b = item_cb(i)
            ldesc[k].wait()
            if i % (B * PCH) == 0:
                pdesc.wait()

            buf = bufs[k]
            pbase = (c % PCH) * C

            @plsc.parallel_loop(0, C * D, L, unroll=8)
            def add_body(j):
                r = j // D
                col = j % D
                plsc.addupdate(
                    buf.at[r, pl.ds(col, L)], pos_v[pbase + r, pl.ds(col, L)]
                )

            if i % (B * PCH) == B * PCH - 1 and i + 1 < NI:
                pdesc = load_pos((i + 1) // (B * PCH))

            sdesc[k] = pltpu.async_copy(
                buf, out_hbm.at[b, pl.ds(r0 + c * C, C), :], ssems[k]
            )

            ni = i + LOOKAHEAD
            if ni < NI:
                nk = ni % NB
                if sdesc[nk] is not None:
                    sdesc[nk].wait()
                ldesc[nk] = load(ni, nk)

        for i in range(max(0, NI - NB), NI):
            sdesc[i % NB].wait()

    return run(x, pos_table)


def kernel(x, pos_table):
    return _kernel_sc(x, pos_table)


# speedup vs baseline: 1.2451x; 1.0126x over previous
import functools

import jax
import jax.numpy as jnp
from jax import lax
from jax.experimental import pallas as pl
from jax.experimental.pallas import tpu as pltpu
from jax.experimental.pallas import tpu_sc as plsc


def _kernel_sc(x, pos_table):
    B, S, D = x.shape
    info = plsc.get_sparse_core_info()
    NC, NS, L = info.num_cores, info.num_subcores, info.num_lanes
    NW = NC * NS
    RW = S // NW
    C = 8
    NCH = RW // C
    NI = B * NCH
    NB = 3

    mesh = plsc.VectorSubcoreMesh(core_axis_name="c", subcore_axis_name="s")

    @functools.partial(
        pl.kernel,
        mesh=mesh,
        out_type=jax.ShapeDtypeStruct((B, S, D), jnp.float32),
        compiler_params=pltpu.CompilerParams(use_tc_tiling_on_sc=True),
        scratch_types=[pltpu.VMEM((RW, D), jnp.float32)]
        + [pltpu.VMEM((C, D), jnp.float32) for _ in range(2 * NB)]
        + [pltpu.SemaphoreType.DMA for _ in range(2 * NB + 1)],
    )
    def run(x_hbm, pos_hbm, out_hbm, pos_v, *rest):
        ibufs = rest[:NB]
        obufs = rest[NB : 2 * NB]
        lsems = rest[2 * NB : 3 * NB]
        ssems = rest[3 * NB : 4 * NB]
        psem = rest[4 * NB]

        wid = lax.axis_index("s") * NC + lax.axis_index("c")
        r0 = wid * RW

        def item_cb(i):
            return i % NCH, i // NCH

        def load(i, k):
            c, b = item_cb(i)
            return pltpu.async_copy(
                x_hbm.at[b, pl.ds(r0 + c * C, C), :], ibufs[k], lsems[k]
            )

        pdesc = pltpu.async_copy(pos_hbm.at[pl.ds(r0, RW), :], pos_v, psem)

        ldesc = [None] * NB
        sdesc = [None] * NB
        for k in range(NB):
            ldesc[k] = load(k, k)

        for i in range(NI):
            k = i % NB
            c, b = item_cb(i)
            ldesc[k].wait()
            if i == 0:
                pdesc.wait()
            if sdesc[k] is not None:
                sdesc[k].wait()

            ibuf = ibufs[k]
            obuf = obufs[k]
            pbase = c * C

            @plsc.parallel_loop(0, C * D, L, unroll=8)
            def add_body(j):
                r = j // D
                col = j % D
                obuf.at[r][pl.ds(col, L)] = (
                    ibuf[r, pl.ds(col, L)] + pos_v[pbase + r, pl.ds(col, L)]
                )

            sdesc[k] = pltpu.async_copy(
                obuf, out_hbm.at[b, pl.ds(r0 + c * C, C), :], ssems[k]
            )

            ni = i + NB
            if ni < NI:
                ldesc[k] = load(ni, k)

        for i in range(max(0, NI - NB), NI):
            sdesc[i % NB].wait()

    return run(x, pos_table)


def kernel(x, pos_table):
    return _kernel_sc(x, pos_table)
